# Initial kernel scaffold; baseline (speedup 1.0000x reference)
import functools

import jax
import jax.numpy as jnp
from jax import lax
from jax.experimental import pallas as pl
from jax.experimental.pallas import tpu as pltpu
from jax.experimental.pallas import tpu_sc as plsc

NC = 2
NS = 16
NW = NC * NS
CHUNK = 80
ZROWS = 125
CNT_W = 16


def _sc_aggregate(src, dst, x):
    n_edges = src.shape[0]
    n_nodes, d = x.shape
    e_per_tile = n_edges // NW
    assert e_per_tile * NW == n_edges and e_per_tile % CHUNK == 0
    n_chunks = e_per_tile // CHUNK
    rows_per_tile = n_nodes // NS
    assert rows_per_tile * NS == n_nodes and rows_per_tile % ZROWS == 0

    mesh = plsc.VectorSubcoreMesh(core_axis_name="c", subcore_axis_name="s")

    @functools.partial(
        pl.kernel,
        out_type=(
            jax.ShapeDtypeStruct((NC, n_nodes, d), jnp.float32),
            jax.ShapeDtypeStruct((NC, n_nodes, CNT_W), jnp.float32),
        ),
        mesh=mesh,
        scratch_types=[
            pltpu.VMEM((CHUNK,), jnp.int32),
            pltpu.VMEM((CHUNK,), jnp.int32),
            pltpu.VMEM((CHUNK, d), jnp.float32),
            pltpu.VMEM((CHUNK, CNT_W), jnp.float32),
            pltpu.VMEM((ZROWS, d), jnp.float32),
            pltpu.VMEM((ZROWS, CNT_W), jnp.float32),
            pltpu.VMEM_SHARED((n_nodes, d), jnp.float32),
            pltpu.VMEM_SHARED((n_nodes, CNT_W), jnp.float32),
        ],
    )
    def k(src_hbm, dst_hbm, x_hbm, agg_hbm, cnt_hbm,
          src_v, dst_v, rows_v, ones_v, zrow_v, zcnt_v, agg_s, cnt_s):
        c = lax.axis_index("c")
        s = lax.axis_index("s")
        wid = s * NC + c

        @pl.loop(0, ZROWS)
        def _(r):
            @pl.loop(0, d, step=16)
            def _(c2):
                zrow_v[r, pl.ds(c2, 16)] = jnp.zeros((16,), jnp.float32)
            zcnt_v[r, :] = jnp.zeros((CNT_W,), jnp.float32)

        @pl.loop(0, CHUNK)
        def _(r):
            ones_v[r, :] = jnp.ones((CNT_W,), jnp.float32)

        @pl.loop(0, rows_per_tile // ZROWS)
        def _(j):
            base = s * rows_per_tile + j * ZROWS
            pltpu.sync_copy(zrow_v, agg_s.at[pl.ds(base, ZROWS)])
            pltpu.sync_copy(zcnt_v, cnt_s.at[pl.ds(base, ZROWS)])

        plsc.subcore_barrier()

        ebase = wid * e_per_tile

        @pl.loop(0, n_chunks)
        def _(i):
            off = ebase + i * CHUNK
            pltpu.sync_copy(src_hbm.at[pl.ds(off, CHUNK)], src_v)
            pltpu.sync_copy(dst_hbm.at[pl.ds(off, CHUNK)], dst_v)
            pltpu.sync_copy(x_hbm.at[src_v], rows_v)
            pltpu.sync_copy(rows_v, agg_s.at[dst_v], add=True)
            pltpu.sync_copy(ones_v, cnt_s.at[dst_v], add=True)

        plsc.subcore_barrier()

        nbase = s * rows_per_tile
        pltpu.sync_copy(agg_s.at[pl.ds(nbase, rows_per_tile)],
                        agg_hbm.at[c].at[pl.ds(nbase, rows_per_tile)])
        pltpu.sync_copy(cnt_s.at[pl.ds(nbase, rows_per_tile)],
                        cnt_hbm.at[c].at[pl.ds(nbase, rows_per_tile)])

    return k(src, dst, x)


def _tc_combine(agg, cnt, x, W_l, b_l, W_r):
    n_nodes, d = x.shape
    blk = 1000
    assert n_nodes % blk == 0

    def body(agg_ref, cnt_ref, x_ref, wl_ref, bl_ref, wr_ref, o_ref):
        a = agg_ref[0] + agg_ref[1]
        c = cnt_ref[0, :, 0:1] + cnt_ref[1, :, 0:1]
        mean = a / jnp.maximum(c, 1.0)
        dn = (((1,), (1,)), ((), ()))
        o_ref[...] = (
            lax.dot_general(mean, wl_ref[...], dn,
                            preferred_element_type=jnp.float32)
            + bl_ref[...]
            + lax.dot_general(x_ref[...], wr_ref[...], dn,
                              preferred_element_type=jnp.float32)
        )

    return pl.pallas_call(
        body,
        grid=(n_nodes // blk,),
        in_specs=[
            pl.BlockSpec((NC, blk, d), lambda i: (0, i, 0)),
            pl.BlockSpec((NC, blk, CNT_W), lambda i: (0, i, 0)),
            pl.BlockSpec((blk, d), lambda i: (i, 0)),
            pl.BlockSpec((d, d), lambda i: (0, 0)),
            pl.BlockSpec((1, d), lambda i: (0, 0)),
            pl.BlockSpec((d, d), lambda i: (0, 0)),
        ],
        out_specs=pl.BlockSpec((blk, d), lambda i: (i, 0)),
        out_shape=jax.ShapeDtypeStruct((n_nodes, d), jnp.float32),
    )(agg, cnt, x, W_l, b_l.reshape(1, d), W_r)


def kernel(x, edge_index, W_l, b_l, W_r):
    ei = edge_index.astype(jnp.int32)
    agg, cnt = _sc_aggregate(ei[0], ei[1], x)
    return _tc_combine(agg, cnt, x, W_l, b_l, W_r)


# speedup vs baseline: 6.2407x; 6.2407x over previous
import dataclasses
import functools

import jax
import jax.numpy as jnp
from jax import lax
from jax.experimental import pallas as pl
from jax.experimental.pallas import tpu as pltpu
from jax.experimental.pallas import tpu_sc as plsc

NC = 2
NS = 16
NW = NC * NS
CHUNK = 80


def _sc_aggregate(src, dst, x):
    n_edges = src.shape[0]
    n_nodes, d = x.shape
    e_per_tile = n_edges // NW
    assert e_per_tile * NW == n_edges and e_per_tile % CHUNK == 0
    n_chunks = e_per_tile // CHUNK
    drain_tiles = 10
    rows_per_tile = n_nodes // drain_tiles
    assert rows_per_tile * drain_tiles == n_nodes
    z_full, z_rem = divmod(rows_per_tile, CHUNK)
    assert z_rem % 8 == 0
    g_rows = -(-n_nodes // d)
    g_rows += (-g_rows) % 8
    assert d == 128

    mesh = plsc.VectorSubcoreMesh(core_axis_name="c", subcore_axis_name="s")
    cp = pltpu.CompilerParams()
    if "needs_layout_passes" in pltpu.CompilerParams.__dataclass_fields__:
        cp = dataclasses.replace(cp, needs_layout_passes=False)

    @functools.partial(
        pl.kernel,
        out_type=(
            jax.ShapeDtypeStruct((NC, n_nodes, d), jnp.float32),
            jax.ShapeDtypeStruct((NW, g_rows, 128), jnp.float32),
        ),
        mesh=mesh,
        compiler_params=cp,
        scratch_types=[
            pltpu.VMEM((CHUNK,), jnp.int32),
            pltpu.VMEM((CHUNK,), jnp.int32),
            pltpu.VMEM((1, CHUNK), jnp.int32),
            pltpu.VMEM((CHUNK, d), jnp.float32),
            pltpu.VMEM((g_rows, 128), jnp.float32),
            pltpu.VMEM_SHARED((n_nodes, d), jnp.float32),
        ],
    )
    def k(src_hbm, dst_hbm, x_hbm, agg_hbm, cntg_hbm,
          src_v, dst_v, dst2_v, rows_v, cntg_v, agg_s):
        c = lax.axis_index("c")
        s = lax.axis_index("s")
        wid = s * NC + c

        @pl.loop(0, CHUNK)
        def _(r):
            @pl.loop(0, d, step=16)
            def _(c2):
                rows_v[r, pl.ds(c2, 16)] = jnp.zeros((16,), jnp.float32)

        @pl.loop(0, g_rows)
        def _(r):
            @pl.loop(0, 128, step=16)
            def _(c2):
                cntg_v[r, pl.ds(c2, 16)] = jnp.zeros((16,), jnp.float32)

        @pl.when(s < drain_tiles)
        def _():
            @pl.loop(0, z_full)
            def _(j):
                base = s * rows_per_tile + j * CHUNK
                pltpu.sync_copy(rows_v, agg_s.at[pl.ds(base, CHUNK)])
            if z_rem:
                base = s * rows_per_tile + z_full * CHUNK
                pltpu.sync_copy(rows_v.at[pl.ds(0, z_rem)],
                                agg_s.at[pl.ds(base, z_rem)])

        plsc.subcore_barrier()

        ebase = wid * e_per_tile
        ones16 = jnp.ones((16,), jnp.float32)

        @pl.loop(0, n_chunks)
        def _(i):
            off = ebase + i * CHUNK
            pltpu.sync_copy(src_hbm.at[pl.ds(off, CHUNK)], src_v)
            pltpu.sync_copy(dst_hbm.at[pl.ds(off, CHUNK)], dst_v)

            @pl.loop(0, CHUNK, step=16)
            def _(j):
                dv = dst_v[pl.ds(j, 16)]
                dst2_v[0, pl.ds(j, 16)] = dv
                plsc.addupdate_scatter(
                    cntg_v, [lax.shift_right_logical(dv, 7),
                             lax.bitwise_and(dv, 127)], ones16)

            pltpu.sync_copy(x_hbm.at[src_v], rows_v)
            pltpu.sync_copy(rows_v, agg_s.at[dst2_v.at[0]], add=True)

        plsc.subcore_barrier()

        pltpu.sync_copy(cntg_v, cntg_hbm.at[wid])

        @pl.when(s < drain_tiles)
        def _():
            @pl.loop(0, z_full)
            def _(j):
                base = s * rows_per_tile + j * CHUNK
                pltpu.sync_copy(agg_s.at[pl.ds(base, CHUNK)], rows_v)
                pltpu.sync_copy(rows_v, agg_hbm.at[c].at[pl.ds(base, CHUNK)])
            if z_rem:
                base = s * rows_per_tile + z_full * CHUNK
                pltpu.sync_copy(agg_s.at[pl.ds(base, z_rem)],
                                rows_v.at[pl.ds(0, z_rem)])
                pltpu.sync_copy(rows_v.at[pl.ds(0, z_rem)],
                                agg_hbm.at[c].at[pl.ds(base, z_rem)])

    return k(src, dst, x)


def _tc_combine(agg, cnt_col, x, W_l, b_l, W_r):
    n_nodes, d = x.shape
    blk = 1000
    assert n_nodes % blk == 0

    def body(agg_ref, cnt_ref, x_ref, wl_ref, bl_ref, wr_ref, o_ref):
        a = agg_ref[0] + agg_ref[1]
        mean = a / jnp.maximum(cnt_ref[...], 1.0)
        dn = (((1,), (1,)), ((), ()))
        o_ref[...] = (
            lax.dot_general(mean, wl_ref[...], dn,
                            preferred_element_type=jnp.float32)
            + bl_ref[...]
            + lax.dot_general(x_ref[...], wr_ref[...], dn,
                              preferred_element_type=jnp.float32)
        )

    return pl.pallas_call(
        body,
        grid=(n_nodes // blk,),
        in_specs=[
            pl.BlockSpec((NC, blk, d), lambda i: (0, i, 0)),
            pl.BlockSpec((blk, 1), lambda i: (i, 0)),
            pl.BlockSpec((blk, d), lambda i: (i, 0)),
            pl.BlockSpec((d, d), lambda i: (0, 0)),
            pl.BlockSpec((1, d), lambda i: (0, 0)),
            pl.BlockSpec((d, d), lambda i: (0, 0)),
        ],
        out_specs=pl.BlockSpec((blk, d), lambda i: (i, 0)),
        out_shape=jax.ShapeDtypeStruct((n_nodes, d), jnp.float32),
    )(agg, cnt_col, x, W_l, b_l.reshape(1, d), W_r)


def kernel(x, edge_index, W_l, b_l, W_r):
    n_nodes, _ = x.shape
    ei = edge_index.astype(jnp.int32)
    agg, cntg = _sc_aggregate(ei[0], ei[1], x)
    cnt_col = cntg.sum(axis=0).reshape(-1)[:n_nodes].reshape(n_nodes, 1)
    return _tc_combine(agg, cnt_col, x, W_l, b_l, W_r)


# speedup vs baseline: 11.5459x; 1.8501x over previous
import dataclasses
import functools

import jax
import jax.numpy as jnp
from jax import lax
from jax.experimental import pallas as pl
from jax.experimental.pallas import tpu as pltpu
from jax.experimental.pallas import tpu_sc as plsc

NC = 2
NS = 16
NW = NC * NS
CHUNK = 80


def _sc_aggregate(src, dst, x):
    n_edges = src.shape[0]
    n_nodes, d = x.shape
    e_per_tile = n_edges // NW
    assert e_per_tile * NW == n_edges and e_per_tile % CHUNK == 0
    n_chunks = e_per_tile // CHUNK
    drain_tiles = 10
    rows_per_tile = n_nodes // drain_tiles
    assert rows_per_tile * drain_tiles == n_nodes
    z_full, z_rem = divmod(rows_per_tile, CHUNK)
    assert z_rem % 8 == 0
    g_rows = -(-n_nodes // d)
    g_rows += (-g_rows) % 8
    assert d == 128

    mesh = plsc.VectorSubcoreMesh(core_axis_name="c", subcore_axis_name="s")
    cp = pltpu.CompilerParams()
    if "needs_layout_passes" in pltpu.CompilerParams.__dataclass_fields__:
        cp = dataclasses.replace(cp, needs_layout_passes=False)

    @functools.partial(
        pl.kernel,
        out_type=(
            jax.ShapeDtypeStruct((NC, n_nodes, d), jnp.float32),
            jax.ShapeDtypeStruct((NW, g_rows, 128), jnp.float32),
        ),
        mesh=mesh,
        compiler_params=cp,
        scratch_types=[
            pltpu.VMEM((2, CHUNK), jnp.int32),
            pltpu.VMEM((e_per_tile,), jnp.int32),
            pltpu.VMEM((2, CHUNK), jnp.int32),
            pltpu.VMEM((CHUNK, d), jnp.float32),
            pltpu.VMEM((CHUNK, d), jnp.float32),
            pltpu.VMEM((g_rows, 128), jnp.float32),
            pltpu.VMEM_SHARED((n_nodes, d), jnp.float32),
            pltpu.SemaphoreType.DMA,
            pltpu.SemaphoreType.DMA,
            pltpu.SemaphoreType.DMA,
            pltpu.SemaphoreType.DMA,
        ],
    )
    def k(src_hbm, dst_hbm, x_hbm, agg_hbm, cntg_hbm,
          srci_v, dstf_v, dst2_v, rows_v, rows_b, cntg_v, agg_s,
          sem_a, sem_b, sem_s0, sem_s1):
        c = lax.axis_index("c")
        s = lax.axis_index("s")
        wid = s * NC + c

        @pl.loop(0, CHUNK)
        def _(r):
            @pl.loop(0, d, step=16)
            def _(c2):
                rows_v[r, pl.ds(c2, 16)] = jnp.zeros((16,), jnp.float32)

        @pl.loop(0, g_rows)
        def _(r):
            @pl.loop(0, 128, step=16)
            def _(c2):
                cntg_v[r, pl.ds(c2, 16)] = jnp.zeros((16,), jnp.float32)

        @pl.when(s < drain_tiles)
        def _():
            @pl.loop(0, z_full)
            def _(j):
                base = s * rows_per_tile + j * CHUNK
                pltpu.sync_copy(rows_v, agg_s.at[pl.ds(base, CHUNK)])
            if z_rem:
                base = s * rows_per_tile + z_full * CHUNK
                pltpu.sync_copy(rows_v.at[pl.ds(0, z_rem)],
                                agg_s.at[pl.ds(base, z_rem)])

        plsc.subcore_barrier()

        ebase = wid * e_per_tile
        ones16 = jnp.ones((16,), jnp.float32)
        assert n_chunks % 2 == 1

        pltpu.sync_copy(dst_hbm.at[pl.ds(ebase, e_per_tile)], dstf_v)

        def load_src(i, slot, sem):
            pltpu.async_copy(src_hbm.at[pl.ds(ebase + i * CHUNK, CHUNK)],
                             srci_v.at[slot], sem)

        def wait_src(i, slot, sem):
            pltpu.make_async_copy(src_hbm.at[pl.ds(ebase + i * CHUNK, CHUNK)],
                                  srci_v.at[slot], sem).wait()

        def start_gather(slot, buf, sem):
            pltpu.async_copy(x_hbm.at[srci_v.at[slot]], buf, sem)

        def wait_gather(slot, buf, sem):
            pltpu.make_async_copy(x_hbm.at[srci_v.at[slot]], buf, sem).wait()

        def prep_dst(i, slot):
            @pl.loop(0, CHUNK, step=16)
            def _(j):
                dv = dstf_v[pl.ds(i * CHUNK + j, 16)]
                dst2_v[slot, pl.ds(j, 16)] = dv
                plsc.addupdate_scatter(
                    cntg_v, [lax.shift_right_logical(dv, 7),
                             lax.bitwise_and(dv, 127)], ones16)

        def scatter(slot, buf):
            pltpu.sync_copy(buf, agg_s.at[dst2_v.at[slot]], add=True)

        load_src(0, 0, sem_s0)
        prep_dst(0, 0)
        wait_src(0, 0, sem_s0)
        start_gather(0, rows_v, sem_a)
        load_src(1, 1, sem_s1)

        @pl.loop(0, n_chunks - 1, step=2)
        def _(i):
            prep_dst(i + 1, 1)
            wait_src(i + 1, 1, sem_s1)
            start_gather(1, rows_b, sem_b)
            wait_gather(0, rows_v, sem_a)
            scatter(0, rows_v)
            load_src(i + 2, 0, sem_s0)
            prep_dst(i + 2, 0)
            wait_src(i + 2, 0, sem_s0)
            start_gather(0, rows_v, sem_a)
            wait_gather(1, rows_b, sem_b)
            scatter(1, rows_b)

            @pl.when(i + 3 < n_chunks)
            def _():
                load_src(i + 3, 1, sem_s1)

        wait_gather(0, rows_v, sem_a)
        scatter(0, rows_v)

        plsc.subcore_barrier()

        pltpu.sync_copy(cntg_v, cntg_hbm.at[wid])

        @pl.when(s < drain_tiles)
        def _():
            @pl.loop(0, z_full)
            def _(j):
                base = s * rows_per_tile + j * CHUNK
                pltpu.sync_copy(agg_s.at[pl.ds(base, CHUNK)], rows_v)
                pltpu.sync_copy(rows_v, agg_hbm.at[c].at[pl.ds(base, CHUNK)])
            if z_rem:
                base = s * rows_per_tile + z_full * CHUNK
                pltpu.sync_copy(agg_s.at[pl.ds(base, z_rem)],
                                rows_v.at[pl.ds(0, z_rem)])
                pltpu.sync_copy(rows_v.at[pl.ds(0, z_rem)],
                                agg_hbm.at[c].at[pl.ds(base, z_rem)])

    return k(src, dst, x)


def _tc_combine(agg, cnt_col, x, W_l, b_l, W_r):
    n_nodes, d = x.shape
    blk = 1000
    assert n_nodes % blk == 0

    def body(agg_ref, cnt_ref, x_ref, wl_ref, bl_ref, wr_ref, o_ref):
        a = agg_ref[0] + agg_ref[1]
        mean = a / jnp.maximum(cnt_ref[...], 1.0)
        dn = (((1,), (1,)), ((), ()))
        o_ref[...] = (
            lax.dot_general(mean, wl_ref[...], dn,
                            preferred_element_type=jnp.float32)
            + bl_ref[...]
            + lax.dot_general(x_ref[...], wr_ref[...], dn,
                              preferred_element_type=jnp.float32)
        )

    return pl.pallas_call(
        body,
        grid=(n_nodes // blk,),
        in_specs=[
            pl.BlockSpec((NC, blk, d), lambda i: (0, i, 0)),
            pl.BlockSpec((blk, 1), lambda i: (i, 0)),
            pl.BlockSpec((blk, d), lambda i: (i, 0)),
            pl.BlockSpec((d, d), lambda i: (0, 0)),
            pl.BlockSpec((1, d), lambda i: (0, 0)),
            pl.BlockSpec((d, d), lambda i: (0, 0)),
        ],
        out_specs=pl.BlockSpec((blk, d), lambda i: (i, 0)),
        out_shape=jax.ShapeDtypeStruct((n_nodes, d), jnp.float32),
    )(agg, cnt_col, x, W_l, b_l.reshape(1, d), W_r)


def kernel(x, edge_index, W_l, b_l, W_r):
    n_nodes, _ = x.shape
    ei = edge_index.astype(jnp.int32)
    agg, cntg = _sc_aggregate(ei[0], ei[1], x)
    cnt_col = cntg.sum(axis=0).reshape(-1)[:n_nodes].reshape(n_nodes, 1)
    return _tc_combine(agg, cnt_col, x, W_l, b_l, W_r)


# speedup vs baseline: 14.8198x; 1.2835x over previous
import dataclasses
import functools

import jax
import jax.numpy as jnp
from jax import lax
from jax.experimental import pallas as pl
from jax.experimental.pallas import tpu as pltpu
from jax.experimental.pallas import tpu_sc as plsc

NC = 2
NS = 16
NW = NC * NS
CHUNK = 80


def _sc_aggregate(src, dst, x):
    n_edges = src.shape[0]
    n_nodes, d = x.shape
    e_per_tile = n_edges // NW
    assert e_per_tile * NW == n_edges and e_per_tile % CHUNK == 0
    n_chunks = e_per_tile // CHUNK
    drain_tiles = 10
    rows_per_tile = n_nodes // drain_tiles
    assert rows_per_tile * drain_tiles == n_nodes
    z_full, z_rem = divmod(rows_per_tile, CHUNK)
    assert z_rem % 8 == 0
    g_rows = -(-n_nodes // d)
    g_rows += (-g_rows) % 8
    assert d == 128

    mesh = plsc.VectorSubcoreMesh(core_axis_name="c", subcore_axis_name="s")
    cp = pltpu.CompilerParams()
    if "needs_layout_passes" in pltpu.CompilerParams.__dataclass_fields__:
        cp = dataclasses.replace(cp, needs_layout_passes=False)

    @functools.partial(
        pl.kernel,
        out_type=(
            jax.ShapeDtypeStruct((NC, n_nodes, d), jnp.float32),
            jax.ShapeDtypeStruct((NW, g_rows, 128), jnp.float32),
        ),
        mesh=mesh,
        compiler_params=cp,
        scratch_types=[
            pltpu.VMEM((3, CHUNK), jnp.int32),
            pltpu.VMEM((3, CHUNK), jnp.int32),
            pltpu.VMEM((6, CHUNK), jnp.int32),
            pltpu.VMEM((CHUNK, d), jnp.float32),
            pltpu.VMEM((CHUNK, d), jnp.float32),
            pltpu.VMEM((CHUNK, d), jnp.float32),
            pltpu.VMEM((g_rows, 128), jnp.float32),
            pltpu.VMEM_SHARED((n_nodes, d), jnp.float32),
            pltpu.SemaphoreType.DMA,
            pltpu.SemaphoreType.DMA,
            pltpu.SemaphoreType.DMA,
            pltpu.SemaphoreType.DMA,
            pltpu.SemaphoreType.DMA,
            pltpu.SemaphoreType.DMA,
            pltpu.SemaphoreType.DMA,
            pltpu.SemaphoreType.DMA,
            pltpu.SemaphoreType.DMA,
        ],
    )
    def k(src_hbm, dst_hbm, x_hbm, agg_hbm, cntg_hbm,
          srci_v, dsti_v, dst2_v, rows_0, rows_1, rows_2, cntg_v, agg_s,
          sem_i0, sem_i1, sem_i2, sem_g0, sem_g1, sem_g2,
          sem_w0, sem_w1, sem_w2):
        c = lax.axis_index("c")
        s = lax.axis_index("s")
        wid = s * NC + c

        @pl.loop(0, CHUNK)
        def _(r):
            @pl.loop(0, d, step=16)
            def _(c2):
                rows_0[r, pl.ds(c2, 16)] = jnp.zeros((16,), jnp.float32)

        @pl.loop(0, g_rows)
        def _(r):
            @pl.loop(0, 128, step=16)
            def _(c2):
                cntg_v[r, pl.ds(c2, 16)] = jnp.zeros((16,), jnp.float32)

        @pl.when(s < drain_tiles)
        def _():
            @pl.loop(0, z_full)
            def _(j):
                base = s * rows_per_tile + j * CHUNK
                pltpu.sync_copy(rows_0, agg_s.at[pl.ds(base, CHUNK)])
            if z_rem:
                base = s * rows_per_tile + z_full * CHUNK
                pltpu.sync_copy(rows_0.at[pl.ds(0, z_rem)],
                                agg_s.at[pl.ds(base, z_rem)])

        plsc.subcore_barrier()

        ebase = wid * e_per_tile
        ones16 = jnp.ones((16,), jnp.float32)
        assert n_chunks % 3 == 2 and n_chunks > 5

        rows = [rows_0, rows_1, rows_2]
        sem_i = [sem_i0, sem_i1, sem_i2]
        sem_g = [sem_g0, sem_g1, sem_g2]
        sem_w = [sem_w0, sem_w1, sem_w2]

        def load_idx(i, b):
            off = ebase + i * CHUNK
            pltpu.async_copy(src_hbm.at[pl.ds(off, CHUNK)],
                             srci_v.at[b], sem_i[b])
            pltpu.async_copy(dst_hbm.at[pl.ds(off, CHUNK)],
                             dsti_v.at[b], sem_i[b])

        def wait_idx(i, b):
            off = ebase + i * CHUNK
            pltpu.make_async_copy(src_hbm.at[pl.ds(off, CHUNK)],
                                  srci_v.at[b], sem_i[b]).wait()
            pltpu.make_async_copy(dst_hbm.at[pl.ds(off, CHUNK)],
                                  dsti_v.at[b], sem_i[b]).wait()

        def prep_dst(b, d2):
            @pl.loop(0, CHUNK, step=16)
            def _(j):
                dv = dsti_v[b, pl.ds(j, 16)]
                dst2_v[d2, pl.ds(j, 16)] = dv
                plsc.addupdate_scatter(
                    cntg_v, [lax.shift_right_logical(dv, 7),
                             lax.bitwise_and(dv, 127)], ones16)

        def start_gather(b):
            pltpu.async_copy(x_hbm.at[srci_v.at[b]], rows[b], sem_g[b])

        def wait_gather(b):
            pltpu.make_async_copy(
                x_hbm.at[srci_v.at[b]], rows[b], sem_g[b]).wait()

        def start_scatter(b, d2):
            pltpu.async_copy(rows[b], agg_s.at[dst2_v.at[d2]], sem_w[b],
                             add=True)

        def wait_scatter(b, d2):
            pltpu.make_async_copy(rows[b], agg_s.at[dst2_v.at[d2]],
                                  sem_w[b]).wait()

        def visit(i, b, q):
            wait_gather(b)
            start_scatter(b, b + 3 * (1 - q))
            load_idx(i, b)
            wait_idx(i, b)
            prep_dst(b, b + 3 * q)
            wait_scatter(b, b + 3 * (1 - q))
            start_gather(b)

        for b in range(3):
            load_idx(b, b)
        for b in range(3):
            wait_idx(b, b)
            prep_dst(b, b)
            start_gather(b)

        assert (n_chunks - 5) % 6 == 0

        @pl.loop(3, n_chunks - 2, step=6)
        def _(k):
            for sub in range(2):
                q = 1 - sub
                for b in range(3):
                    visit(k + 3 * sub + b, b, q)

        visit(n_chunks - 2, 0, 1)
        visit(n_chunks - 1, 1, 1)
        wait_gather(2)
        start_scatter(2, 2)
        wait_gather(0)
        start_scatter(0, 3)
        wait_gather(1)
        start_scatter(1, 4)
        wait_scatter(2, 2)
        wait_scatter(0, 3)
        wait_scatter(1, 4)

        plsc.subcore_barrier()

        pltpu.sync_copy(cntg_v, cntg_hbm.at[wid])

        @pl.when(s < drain_tiles)
        def _():
            @pl.loop(0, z_full)
            def _(j):
                base = s * rows_per_tile + j * CHUNK
                pltpu.sync_copy(agg_s.at[pl.ds(base, CHUNK)], rows_0)
                pltpu.sync_copy(rows_0, agg_hbm.at[c].at[pl.ds(base, CHUNK)])
            if z_rem:
                base = s * rows_per_tile + z_full * CHUNK
                pltpu.sync_copy(agg_s.at[pl.ds(base, z_rem)],
                                rows_0.at[pl.ds(0, z_rem)])
                pltpu.sync_copy(rows_0.at[pl.ds(0, z_rem)],
                                agg_hbm.at[c].at[pl.ds(base, z_rem)])

    return k(src, dst, x)


def _tc_combine(agg, cnt_col, x, W_l, b_l, W_r):
    n_nodes, d = x.shape
    blk = 1000
    assert n_nodes % blk == 0

    def body(agg_ref, cnt_ref, x_ref, wl_ref, bl_ref, wr_ref, o_ref):
        a = agg_ref[0] + agg_ref[1]
        mean = a / jnp.maximum(cnt_ref[...], 1.0)
        dn = (((1,), (1,)), ((), ()))
        o_ref[...] = (
            lax.dot_general(mean, wl_ref[...], dn,
                            preferred_element_type=jnp.float32)
            + bl_ref[...]
            + lax.dot_general(x_ref[...], wr_ref[...], dn,
                              preferred_element_type=jnp.float32)
        )

    return pl.pallas_call(
        body,
        grid=(n_nodes // blk,),
        in_specs=[
            pl.BlockSpec((NC, blk, d), lambda i: (0, i, 0)),
            pl.BlockSpec((blk, 1), lambda i: (i, 0)),
            pl.BlockSpec((blk, d), lambda i: (i, 0)),
            pl.BlockSpec((d, d), lambda i: (0, 0)),
            pl.BlockSpec((1, d), lambda i: (0, 0)),
            pl.BlockSpec((d, d), lambda i: (0, 0)),
        ],
        out_specs=pl.BlockSpec((blk, d), lambda i: (i, 0)),
        out_shape=jax.ShapeDtypeStruct((n_nodes, d), jnp.float32),
    )(agg, cnt_col, x, W_l, b_l.reshape(1, d), W_r)


def kernel(x, edge_index, W_l, b_l, W_r):
    n_nodes, _ = x.shape
    ei = edge_index.astype(jnp.int32)
    agg, cntg = _sc_aggregate(ei[0], ei[1], x)
    cnt_col = cntg.sum(axis=0).reshape(-1)[:n_nodes].reshape(n_nodes, 1)
    return _tc_combine(agg, cnt_col, x, W_l, b_l, W_r)


# speedup vs baseline: 14.9228x; 1.0070x over previous
import dataclasses
import functools

import jax
import jax.numpy as jnp
from jax import lax
from jax.experimental import pallas as pl
from jax.experimental.pallas import tpu as pltpu
from jax.experimental.pallas import tpu_sc as plsc

NC = 2
NS = 16
NW = NC * NS
CHUNK = 80


def _sc_aggregate(src, dst, x):
    n_edges = src.shape[0]
    n_nodes, d = x.shape
    e_per_tile = n_edges // NW
    assert e_per_tile * NW == n_edges and e_per_tile % CHUNK == 0
    n_chunks = e_per_tile // CHUNK
    drain_tiles = 10
    rows_per_tile = n_nodes // drain_tiles
    assert rows_per_tile * drain_tiles == n_nodes
    z_full, z_rem = divmod(rows_per_tile, CHUNK)
    assert z_rem % 8 == 0
    g_rows = -(-n_nodes // d)
    g_rows += (-g_rows) % 8
    assert d == 128

    mesh = plsc.VectorSubcoreMesh(core_axis_name="c", subcore_axis_name="s")
    cp = pltpu.CompilerParams()
    if "needs_layout_passes" in pltpu.CompilerParams.__dataclass_fields__:
        cp = dataclasses.replace(cp, needs_layout_passes=False)

    @functools.partial(
        pl.kernel,
        out_type=(
            jax.ShapeDtypeStruct((NC, n_nodes, d), jnp.float32),
            jax.ShapeDtypeStruct((NW, g_rows, 128), jnp.float32),
        ),
        mesh=mesh,
        compiler_params=cp,
        scratch_types=[
            pltpu.VMEM((3, CHUNK), jnp.int32),
            pltpu.VMEM((3, CHUNK), jnp.int32),
            pltpu.VMEM((6, CHUNK), jnp.int32),
            pltpu.VMEM((CHUNK, d), jnp.float32),
            pltpu.VMEM((CHUNK, d), jnp.float32),
            pltpu.VMEM((CHUNK, d), jnp.float32),
            pltpu.VMEM((g_rows, 128), jnp.float32),
            pltpu.VMEM_SHARED((n_nodes, d), jnp.float32),
            pltpu.SemaphoreType.DMA,
            pltpu.SemaphoreType.DMA,
            pltpu.SemaphoreType.DMA,
            pltpu.SemaphoreType.DMA,
            pltpu.SemaphoreType.DMA,
            pltpu.SemaphoreType.DMA,
            pltpu.SemaphoreType.DMA,
            pltpu.SemaphoreType.DMA,
            pltpu.SemaphoreType.DMA,
        ],
    )
    def k(src_hbm, dst_hbm, x_hbm, agg_hbm, cntg_hbm,
          srci_v, dsti_v, dst2_v, rows_0, rows_1, rows_2, cntg_v, agg_s,
          sem_i0, sem_i1, sem_i2, sem_g0, sem_g1, sem_g2,
          sem_w0, sem_w1, sem_w2):
        c = lax.axis_index("c")
        s = lax.axis_index("s")
        wid = s * NC + c

        assert g_rows == CHUNK
        @pl.loop(0, g_rows)
        def _(r):
            @pl.loop(0, 128, step=16)
            def _(c2):
                cntg_v[r, pl.ds(c2, 16)] = jnp.zeros((16,), jnp.float32)

        @pl.when(s < drain_tiles)
        def _():
            @pl.loop(0, z_full)
            def _(j):
                base = s * rows_per_tile + j * CHUNK
                pltpu.sync_copy(cntg_v, agg_s.at[pl.ds(base, CHUNK)])
            if z_rem:
                base = s * rows_per_tile + z_full * CHUNK
                pltpu.sync_copy(cntg_v.at[pl.ds(0, z_rem)],
                                agg_s.at[pl.ds(base, z_rem)])

        ebase = wid * e_per_tile
        ones16 = jnp.ones((16,), jnp.float32)
        assert n_chunks % 3 == 2 and n_chunks > 5

        rows = [rows_0, rows_1, rows_2]
        sem_i = [sem_i0, sem_i1, sem_i2]
        sem_g = [sem_g0, sem_g1, sem_g2]
        sem_w = [sem_w0, sem_w1, sem_w2]

        def load_idx(i, b):
            off = ebase + i * CHUNK
            pltpu.async_copy(src_hbm.at[pl.ds(off, CHUNK)],
                             srci_v.at[b], sem_i[b])
            pltpu.async_copy(dst_hbm.at[pl.ds(off, CHUNK)],
                             dsti_v.at[b], sem_i[b])

        def wait_idx(i, b):
            off = ebase + i * CHUNK
            pltpu.make_async_copy(src_hbm.at[pl.ds(off, CHUNK)],
                                  srci_v.at[b], sem_i[b]).wait()
            pltpu.make_async_copy(dst_hbm.at[pl.ds(off, CHUNK)],
                                  dsti_v.at[b], sem_i[b]).wait()

        def prep_dst(b, d2):
            @pl.loop(0, CHUNK, step=16)
            def _(j):
                dv = dsti_v[b, pl.ds(j, 16)]
                dst2_v[d2, pl.ds(j, 16)] = dv
                plsc.addupdate_scatter(
                    cntg_v, [lax.shift_right_logical(dv, 7),
                             lax.bitwise_and(dv, 127)], ones16)

        def start_gather(b):
            pltpu.async_copy(x_hbm.at[srci_v.at[b]], rows[b], sem_g[b])

        def wait_gather(b):
            pltpu.make_async_copy(
                x_hbm.at[srci_v.at[b]], rows[b], sem_g[b]).wait()

        def start_scatter(b, d2):
            pltpu.async_copy(rows[b], agg_s.at[dst2_v.at[d2]], sem_w[b],
                             add=True)

        def wait_scatter(b, d2):
            pltpu.make_async_copy(rows[b], agg_s.at[dst2_v.at[d2]],
                                  sem_w[b]).wait()

        def visit(i, b, q):
            wait_gather(b)
            start_scatter(b, b + 3 * (1 - q))
            load_idx(i, b)
            wait_idx(i, b)
            prep_dst(b, b + 3 * q)
            wait_scatter(b, b + 3 * (1 - q))
            start_gather(b)

        for b in range(3):
            load_idx(b, b)
        for b in range(3):
            wait_idx(b, b)
            prep_dst(b, b)
            start_gather(b)

        plsc.subcore_barrier()

        assert (n_chunks - 5) % 6 == 0

        @pl.loop(3, n_chunks - 2, step=6)
        def _(k):
            for sub in range(2):
                q = 1 - sub
                for b in range(3):
                    visit(k + 3 * sub + b, b, q)

        visit(n_chunks - 2, 0, 1)
        visit(n_chunks - 1, 1, 1)
        wait_gather(2)
        start_scatter(2, 2)
        wait_gather(0)
        start_scatter(0, 3)
        wait_gather(1)
        start_scatter(1, 4)
        wait_scatter(2, 2)
        wait_scatter(0, 3)
        wait_scatter(1, 4)

        plsc.subcore_barrier()

        pltpu.sync_copy(cntg_v, cntg_hbm.at[wid])

        @pl.when(s < drain_tiles)
        def _():
            @pl.loop(0, z_full)
            def _(j):
                base = s * rows_per_tile + j * CHUNK
                pltpu.sync_copy(agg_s.at[pl.ds(base, CHUNK)], rows_0)
                pltpu.sync_copy(rows_0, agg_hbm.at[c].at[pl.ds(base, CHUNK)])
            if z_rem:
                base = s * rows_per_tile + z_full * CHUNK
                pltpu.sync_copy(agg_s.at[pl.ds(base, z_rem)],
                                rows_0.at[pl.ds(0, z_rem)])
                pltpu.sync_copy(rows_0.at[pl.ds(0, z_rem)],
                                agg_hbm.at[c].at[pl.ds(base, z_rem)])

    return k(src, dst, x)


def _tc_self(x, W_r, b_l):
    n_nodes, d = x.shape
    blk = 1000
    assert n_nodes % blk == 0

    def body(x_ref, wr_ref, bl_ref, o_ref):
        dn = (((1,), (1,)), ((), ()))
        o_ref[...] = lax.dot_general(
            x_ref[...], wr_ref[...], dn,
            preferred_element_type=jnp.float32) + bl_ref[...]

    return pl.pallas_call(
        body,
        grid=(n_nodes // blk,),
        in_specs=[
            pl.BlockSpec((blk, d), lambda i: (i, 0)),
            pl.BlockSpec((d, d), lambda i: (0, 0)),
            pl.BlockSpec((1, d), lambda i: (0, 0)),
        ],
        out_specs=pl.BlockSpec((blk, d), lambda i: (i, 0)),
        out_shape=jax.ShapeDtypeStruct((n_nodes, d), jnp.float32),
    )(x, W_r, b_l.reshape(1, d))


def _tc_final(agg, cnt_col, self_part, W_l):
    n_nodes, d = self_part.shape
    blk = 1000
    assert n_nodes % blk == 0

    def body(agg_ref, cnt_ref, sp_ref, wl_ref, o_ref):
        a = agg_ref[0] + agg_ref[1]
        mean = a / jnp.maximum(cnt_ref[...], 1.0)
        dn = (((1,), (1,)), ((), ()))
        o_ref[...] = lax.dot_general(
            mean, wl_ref[...], dn,
            preferred_element_type=jnp.float32) + sp_ref[...]

    return pl.pallas_call(
        body,
        grid=(n_nodes // blk,),
        in_specs=[
            pl.BlockSpec((NC, blk, d), lambda i: (0, i, 0)),
            pl.BlockSpec((blk, 1), lambda i: (i, 0)),
            pl.BlockSpec((blk, d), lambda i: (i, 0)),
            pl.BlockSpec((d, d), lambda i: (0, 0)),
        ],
        out_specs=pl.BlockSpec((blk, d), lambda i: (i, 0)),
        out_shape=jax.ShapeDtypeStruct((n_nodes, d), jnp.float32),
    )(agg, cnt_col, self_part, W_l)


def kernel(x, edge_index, W_l, b_l, W_r):
    n_nodes, _ = x.shape
    ei = edge_index.astype(jnp.int32)
    agg, cntg = _sc_aggregate(ei[0], ei[1], x)
    self_part = _tc_self(x, W_r, b_l)
    cnt_col = cntg.sum(axis=0).reshape(-1)[:n_nodes].reshape(n_nodes, 1)
    return _tc_final(agg, cnt_col, self_part, W_l)


# speedup vs baseline: 15.5475x; 1.0419x over previous
import dataclasses
import functools

import jax
import jax.numpy as jnp
from jax import lax
from jax.experimental import pallas as pl
from jax.experimental.pallas import tpu as pltpu
from jax.experimental.pallas import tpu_sc as plsc

NC = 2
NS = 16
NW = NC * NS
CHUNK = 80


def _sc_aggregate(src, dst, x):
    n_edges = src.shape[0]
    n_nodes, d = x.shape
    e_per_tile = n_edges // NW
    assert e_per_tile * NW == n_edges and e_per_tile % CHUNK == 0
    n_chunks = e_per_tile // CHUNK
    d_base = (n_nodes // NS) // 8 * 8
    rem = n_nodes - NS * d_base
    assert rem % 8 == 0 and rem <= CHUNK
    extra_base = NS * d_base
    z_full, z_rem = divmod(d_base, CHUNK)
    assert z_rem % 8 == 0
    g_rows = -(-n_nodes // d)
    g_rows += (-g_rows) % 8
    assert d == 128

    mesh = plsc.VectorSubcoreMesh(core_axis_name="c", subcore_axis_name="s")
    cp = pltpu.CompilerParams()
    if "needs_layout_passes" in pltpu.CompilerParams.__dataclass_fields__:
        cp = dataclasses.replace(cp, needs_layout_passes=False)

    @functools.partial(
        pl.kernel,
        out_type=(
            jax.ShapeDtypeStruct((NC, n_nodes, d), jnp.float32),
            jax.ShapeDtypeStruct((NW, g_rows, 128), jnp.float32),
        ),
        mesh=mesh,
        compiler_params=cp,
        scratch_types=[
            pltpu.VMEM((3, CHUNK), jnp.int32),
            pltpu.VMEM((3, CHUNK), jnp.int32),
            pltpu.VMEM((6, CHUNK), jnp.int32),
            pltpu.VMEM((CHUNK, d), jnp.float32),
            pltpu.VMEM((CHUNK, d), jnp.float32),
            pltpu.VMEM((CHUNK, d), jnp.float32),
            pltpu.VMEM((g_rows, 128), jnp.float32),
            pltpu.VMEM_SHARED((n_nodes, d), jnp.float32),
            pltpu.SemaphoreType.DMA,
            pltpu.SemaphoreType.DMA,
            pltpu.SemaphoreType.DMA,
            pltpu.SemaphoreType.DMA,
            pltpu.SemaphoreType.DMA,
            pltpu.SemaphoreType.DMA,
            pltpu.SemaphoreType.DMA,
            pltpu.SemaphoreType.DMA,
            pltpu.SemaphoreType.DMA,
        ],
    )
    def k(src_hbm, dst_hbm, x_hbm, agg_hbm, cntg_hbm,
          srci_v, dsti_v, dst2_v, rows_0, rows_1, rows_2, cntg_v, agg_s,
          sem_i0, sem_i1, sem_i2, sem_g0, sem_g1, sem_g2,
          sem_w0, sem_w1, sem_w2):
        c = lax.axis_index("c")
        s = lax.axis_index("s")
        wid = s * NC + c

        assert g_rows == CHUNK
        @pl.loop(0, g_rows)
        def _(r):
            @pl.loop(0, 128, step=16)
            def _(c2):
                cntg_v[r, pl.ds(c2, 16)] = jnp.zeros((16,), jnp.float32)

        @pl.loop(0, z_full)
        def _(j):
            base = s * d_base + j * CHUNK
            pltpu.sync_copy(cntg_v, agg_s.at[pl.ds(base, CHUNK)])
        if z_rem:
            base = s * d_base + z_full * CHUNK
            pltpu.sync_copy(cntg_v.at[pl.ds(0, z_rem)],
                            agg_s.at[pl.ds(base, z_rem)])

        if rem:
            @pl.when(s == 0)
            def _():
                pltpu.sync_copy(cntg_v.at[pl.ds(0, rem)],
                                agg_s.at[pl.ds(extra_base, rem)])

        ebase = wid * e_per_tile
        ones16 = jnp.ones((16,), jnp.float32)
        assert n_chunks % 3 == 2 and n_chunks > 5

        rows = [rows_0, rows_1, rows_2]
        sem_i = [sem_i0, sem_i1, sem_i2]
        sem_g = [sem_g0, sem_g1, sem_g2]
        sem_w = [sem_w0, sem_w1, sem_w2]

        def load_idx(i, b):
            off = ebase + i * CHUNK
            pltpu.async_copy(src_hbm.at[pl.ds(off, CHUNK)],
                             srci_v.at[b], sem_i[b])
            pltpu.async_copy(dst_hbm.at[pl.ds(off, CHUNK)],
                             dsti_v.at[b], sem_i[b])

        def wait_idx(i, b):
            off = ebase + i * CHUNK
            pltpu.make_async_copy(src_hbm.at[pl.ds(off, CHUNK)],
                                  srci_v.at[b], sem_i[b]).wait()
            pltpu.make_async_copy(dst_hbm.at[pl.ds(off, CHUNK)],
                                  dsti_v.at[b], sem_i[b]).wait()

        def prep_dst(b, d2):
            @pl.loop(0, CHUNK, step=16)
            def _(j):
                dv = dsti_v[b, pl.ds(j, 16)]
                dst2_v[d2, pl.ds(j, 16)] = dv
                plsc.addupdate_scatter(
                    cntg_v, [lax.shift_right_logical(dv, 7),
                             lax.bitwise_and(dv, 127)], ones16)

        def start_gather(b):
            pltpu.async_copy(x_hbm.at[srci_v.at[b]], rows[b], sem_g[b])

        def wait_gather(b):
            pltpu.make_async_copy(
                x_hbm.at[srci_v.at[b]], rows[b], sem_g[b]).wait()

        def start_scatter(b, d2):
            pltpu.async_copy(rows[b], agg_s.at[dst2_v.at[d2]], sem_w[b],
                             add=True)

        def wait_scatter(b, d2):
            pltpu.make_async_copy(rows[b], agg_s.at[dst2_v.at[d2]],
                                  sem_w[b]).wait()

        def visit(i, b, q):
            wait_gather(b)
            start_scatter(b, b + 3 * (1 - q))
            load_idx(i, b)
            wait_idx(i, b)
            prep_dst(b, b + 3 * q)
            wait_scatter(b, b + 3 * (1 - q))
            start_gather(b)

        for b in range(3):
            load_idx(b, b)
        for b in range(3):
            wait_idx(b, b)
            prep_dst(b, b)
            start_gather(b)

        plsc.subcore_barrier()

        assert (n_chunks - 5) % 6 == 0

        @pl.loop(3, n_chunks - 2, step=6)
        def _(k):
            for sub in range(2):
                q = 1 - sub
                for b in range(3):
                    visit(k + 3 * sub + b, b, q)

        visit(n_chunks - 2, 0, 1)
        visit(n_chunks - 1, 1, 1)
        wait_gather(2)
        start_scatter(2, 2)
        wait_gather(0)
        start_scatter(0, 3)
        wait_gather(1)
        start_scatter(1, 4)
        wait_scatter(2, 2)
        wait_scatter(0, 3)
        wait_scatter(1, 4)

        plsc.subcore_barrier()

        pltpu.sync_copy(cntg_v, cntg_hbm.at[wid])

        nb = s * d_base
        pltpu.sync_copy(agg_s.at[pl.ds(nb, d_base)],
                        agg_hbm.at[c].at[pl.ds(nb, d_base)])

        if rem:
            @pl.when(s == 0)
            def _():
                pltpu.sync_copy(agg_s.at[pl.ds(extra_base, rem)],
                                agg_hbm.at[c].at[pl.ds(extra_base, rem)])

    return k(src, dst, x)


def _tc_self(x, W_r, b_l):
    n_nodes, d = x.shape
    blk = 1000
    assert n_nodes % blk == 0

    def body(x_ref, wr_ref, bl_ref, o_ref):
        dn = (((1,), (1,)), ((), ()))
        o_ref[...] = lax.dot_general(
            x_ref[...], wr_ref[...], dn,
            preferred_element_type=jnp.float32) + bl_ref[...]

    return pl.pallas_call(
        body,
        grid=(n_nodes // blk,),
        in_specs=[
            pl.BlockSpec((blk, d), lambda i: (i, 0)),
            pl.BlockSpec((d, d), lambda i: (0, 0)),
            pl.BlockSpec((1, d), lambda i: (0, 0)),
        ],
        out_specs=pl.BlockSpec((blk, d), lambda i: (i, 0)),
        out_shape=jax.ShapeDtypeStruct((n_nodes, d), jnp.float32),
    )(x, W_r, b_l.reshape(1, d))


def _tc_final(agg, cnt_col, self_part, W_l):
    n_nodes, d = self_part.shape
    blk = 1000
    assert n_nodes % blk == 0

    def body(agg_ref, cnt_ref, sp_ref, wl_ref, o_ref):
        a = agg_ref[0] + agg_ref[1]
        mean = a / jnp.maximum(cnt_ref[...], 1.0)
        dn = (((1,), (1,)), ((), ()))
        o_ref[...] = lax.dot_general(
            mean, wl_ref[...], dn,
            preferred_element_type=jnp.float32) + sp_ref[...]

    return pl.pallas_call(
        body,
        grid=(n_nodes // blk,),
        in_specs=[
            pl.BlockSpec((NC, blk, d), lambda i: (0, i, 0)),
            pl.BlockSpec((blk, 1), lambda i: (i, 0)),
            pl.BlockSpec((blk, d), lambda i: (i, 0)),
            pl.BlockSpec((d, d), lambda i: (0, 0)),
        ],
        out_specs=pl.BlockSpec((blk, d), lambda i: (i, 0)),
        out_shape=jax.ShapeDtypeStruct((n_nodes, d), jnp.float32),
    )(agg, cnt_col, self_part, W_l)


def kernel(x, edge_index, W_l, b_l, W_r):
    n_nodes, _ = x.shape
    ei = edge_index.astype(jnp.int32)
    agg, cntg = _sc_aggregate(ei[0], ei[1], x)
    self_part = _tc_self(x, W_r, b_l)
    cnt_col = cntg.sum(axis=0).reshape(-1)[:n_nodes].reshape(n_nodes, 1)
    return _tc_final(agg, cnt_col, self_part, W_l)
